# Initial kernel scaffold; baseline (speedup 1.0000x reference)
import jax
import jax.numpy as jnp
from jax.experimental import pallas as pl


def kernel(x, input_q, W, b):
    raise NotImplementedError("write your pallas kernel here")


# speedup vs baseline: 1040.3467x; 1040.3467x over previous
import functools

import numpy as np
import jax
import jax.numpy as jnp
from jax import lax
from jax.experimental import pallas as pl
from jax.experimental.pallas import tpu as pltpu
from jax.experimental.pallas import tpu_sc as plsc

_B = 16384
_D = 128
_Q = 200
_K = 99
_KP = 128

_KNOT0 = float(np.float32(1.0) / np.float32(100.0))
_KNOT98 = float(np.float32(99.0) / np.float32(100.0))
_EPS = np.float32(1e-4)

_FK0 = np.float32(1.0) / np.float32(100.0)
_FK1 = np.float32(2.0) / np.float32(100.0)
_FK2 = np.float32(3.0) / np.float32(100.0)
_FK97 = np.float32(98.0) / np.float32(100.0)
_FK98 = np.float32(99.0) / np.float32(100.0)
_BRN = float(np.log(
    (np.float32(1.0) - _FK97 + _EPS) / (np.float32(1.0) - _FK98 + _EPS) + _EPS))
_BLN = float(np.log((_FK2 + _EPS) / (_FK1 + _EPS) + _EPS))
_LOG_1M_FK98 = float(np.log(np.float32(1.0) - _FK98))
_LOG_FK0 = float(np.log(_FK0))

_RB = 512


def _tc_body(x_ref, q_ref, w_ref, b_ref, qo_ref, fb_ref):
    x = x_ref[...]
    z = jnp.dot(x, w_ref[...], preferred_element_type=jnp.float32) + b_ref[...]
    s = jax.nn.softplus(z)
    rows_i = lax.broadcasted_iota(jnp.int32, (_KP, _KP), 0)
    cols_i = lax.broadcasted_iota(jnp.int32, (_KP, _KP), 1)
    tri = ((rows_i <= cols_i) & (rows_i < _K)).astype(jnp.float32)
    qo = jnp.dot(s, tri, preferred_element_type=jnp.float32)
    qo_ref[...] = qo

    q = q_ref[...]
    right_a = (qo[:, 97:98] - qo[:, 98:99]) / _BRN
    right_b = -right_a * _LOG_1M_FK98 + qo[:, 98:99]
    right_cal = jnp.log(1.0 - q) * right_a + right_b
    left_a = (qo[:, 1:2] - qo[:, 0:1]) / _BLN
    left_b = -left_a * _LOG_FK0 + qo[:, 0:1]
    left_cal = jnp.log(q) * left_a + left_b
    fb_ref[...] = jnp.where(q < _KNOT0, left_cal, right_cal)


_NW = 32
_RW = _B // _NW
_CH = 64
_NCHUNK = _RW // _CH
_VEC = _CH * _Q // 16


def _sc_kernel(q_hbm, fb_hbm, qo_hbm, out_hbm, qbuf, fbbuf, qobuf, obuf):
    cid = lax.axis_index("c")
    sid = lax.axis_index("s")
    wid = sid * 2 + cid
    row0 = wid * _RW

    def chunk_body(ci, _):
        rbase = row0 + ci * _CH
        pltpu.sync_copy(q_hbm.at[pl.ds(rbase * _Q, _CH * _Q)], qbuf)
        pltpu.sync_copy(fb_hbm.at[pl.ds(rbase * _Q, _CH * _Q)], fbbuf)
        pltpu.sync_copy(qo_hbm.at[pl.ds(rbase * _KP, _CH * _KP)], qobuf)

        def vec_body(j, _):
            base = j * 16
            pos = lax.iota(jnp.int32, 16) + base
            row = lax.shift_right_logical(pos * 5243, 20)
            qv = qbuf[pl.ds(base, 16)]
            t = qv * 100.0
            c = t.astype(jnp.int32)
            kc = (c + 1).astype(jnp.float32) / 100.0
            c = jnp.where((c <= 98) & (kc <= qv), c + 1, c)
            kcm = c.astype(jnp.float32) / 100.0
            c = jnp.where((c >= 1) & (kcm > qv), c - 1, c)
            r = jnp.minimum(c, 98)
            l = jnp.maximum(r - 1, 0)
            kl = (l + 1).astype(jnp.float32) / 100.0
            kr = (r + 1).astype(jnp.float32) / 100.0
            ratio = (qv - kl) / (kr - kl + 0.001)
            off = row * _KP
            yl = plsc.load_gather(qobuf, [off + l])
            yr = plsc.load_gather(qobuf, [off + r])
            center = yl + ratio * (yr - yl)
            fbv = fbbuf[pl.ds(base, 16)]
            tail = (qv < _KNOT0) | (qv > _KNOT98)
            obuf[pl.ds(base, 16)] = jnp.where(tail, fbv, center)
            return 0

        lax.fori_loop(0, _VEC, vec_body, 0)
        pltpu.sync_copy(obuf, out_hbm.at[pl.ds(rbase * _Q, _CH * _Q)])
        return 0

    lax.fori_loop(0, _NCHUNK, chunk_body, 0)


def kernel(x, input_q, W, b):
    assert x.shape == (_B, _D) and input_q.shape == (_B, _Q)
    w_pad = jnp.zeros((_D, _KP), jnp.float32).at[:, :_K].set(W)
    b_pad = jnp.zeros((1, _KP), jnp.float32).at[0, :_K].set(b)

    qo, fb = pl.pallas_call(
        _tc_body,
        grid=(_B // _RB,),
        in_specs=[
            pl.BlockSpec((_RB, _D), lambda i: (i, 0)),
            pl.BlockSpec((_RB, _Q), lambda i: (i, 0)),
            pl.BlockSpec((_D, _KP), lambda i: (0, 0)),
            pl.BlockSpec((1, _KP), lambda i: (0, 0)),
        ],
        out_specs=[
            pl.BlockSpec((_RB, _KP), lambda i: (i, 0)),
            pl.BlockSpec((_RB, _Q), lambda i: (i, 0)),
        ],
        out_shape=[
            jax.ShapeDtypeStruct((_B, _KP), jnp.float32),
            jax.ShapeDtypeStruct((_B, _Q), jnp.float32),
        ],
    )(x, input_q, w_pad, b_pad)

    sc = functools.partial(
        pl.kernel,
        mesh=plsc.VectorSubcoreMesh(
            core_axis_name="c", subcore_axis_name="s",
            num_cores=2, num_subcores=16),
        out_type=jax.ShapeDtypeStruct((_B * _Q,), jnp.float32),
        scratch_types=[
            pltpu.VMEM((_CH * _Q,), jnp.float32),
            pltpu.VMEM((_CH * _Q,), jnp.float32),
            pltpu.VMEM((_CH * _KP,), jnp.float32),
            pltpu.VMEM((_CH * _Q,), jnp.float32),
        ],
        compiler_params=pltpu.CompilerParams(needs_layout_passes=False),
    )(_sc_kernel)

    out_flat = sc(input_q.reshape(-1), fb.reshape(-1), qo.reshape(-1))
    return out_flat.reshape(_B, _Q)


# speedup vs baseline: 1901.7942x; 1.8280x over previous
import functools

import numpy as np
import jax
import jax.numpy as jnp
from jax import lax
from jax.experimental import pallas as pl
from jax.experimental.pallas import tpu as pltpu
from jax.experimental.pallas import tpu_sc as plsc

_B = 16384
_D = 128
_Q = 200
_K = 99
_KP = 128

_KNOT0 = float(np.float32(1.0) / np.float32(100.0))
_KNOT98 = float(np.float32(99.0) / np.float32(100.0))
_EPS = np.float32(1e-4)

_FK0 = np.float32(1.0) / np.float32(100.0)
_FK1 = np.float32(2.0) / np.float32(100.0)
_FK2 = np.float32(3.0) / np.float32(100.0)
_FK97 = np.float32(98.0) / np.float32(100.0)
_FK98 = np.float32(99.0) / np.float32(100.0)
_BRN = float(np.log(
    (np.float32(1.0) - _FK97 + _EPS) / (np.float32(1.0) - _FK98 + _EPS) + _EPS))
_BLN = float(np.log((_FK2 + _EPS) / (_FK1 + _EPS) + _EPS))
_LOG_1M_FK98 = float(np.log(np.float32(1.0) - _FK98))
_LOG_FK0 = float(np.log(_FK0))

_RB = 512


def _tc_body(x_ref, q_ref, w_ref, b_ref, qo_ref, fb_ref):
    x = x_ref[...]
    z = jnp.dot(x, w_ref[...], preferred_element_type=jnp.float32) + b_ref[...]
    s = jax.nn.softplus(z)
    rows_i = lax.broadcasted_iota(jnp.int32, (_KP, _KP), 0)
    cols_i = lax.broadcasted_iota(jnp.int32, (_KP, _KP), 1)
    tri = ((rows_i <= cols_i) & (rows_i < _K)).astype(jnp.float32)
    qo = jnp.dot(s, tri, preferred_element_type=jnp.float32)
    qo_ref[...] = qo

    q = q_ref[...]
    right_a = (qo[:, 97:98] - qo[:, 98:99]) / _BRN
    right_b = -right_a * _LOG_1M_FK98 + qo[:, 98:99]
    right_cal = jnp.log(1.0 - q) * right_a + right_b
    left_a = (qo[:, 1:2] - qo[:, 0:1]) / _BLN
    left_b = -left_a * _LOG_FK0 + qo[:, 0:1]
    left_cal = jnp.log(q) * left_a + left_b
    fb_ref[...] = jnp.where(q < _KNOT0, left_cal, right_cal)


_NW = 32
_RW = _B // _NW
_CH = 64
_NCHUNK = _RW // _CH
_VEC = _CH * _Q // 16


def _sc_kernel(q_hbm, fb_hbm, qo_hbm, out_hbm, qbuf, fbbuf, qobuf, obuf):
    cid = lax.axis_index("c")
    sid = lax.axis_index("s")
    wid = sid * 2 + cid
    row0 = wid * _RW
    iot = lax.iota(jnp.int32, 16)

    def chunk_body(ci, _):
        rbase = row0 + ci * _CH
        pltpu.sync_copy(q_hbm.at[pl.ds(rbase * _Q, _CH * _Q)], qbuf)
        pltpu.sync_copy(fb_hbm.at[pl.ds(rbase * _Q, _CH * _Q)], fbbuf)
        pltpu.sync_copy(qo_hbm.at[pl.ds(rbase * _KP, _CH * _KP)], qobuf)

        @plsc.parallel_loop(0, _VEC, unroll=8)
        def vec_body(j):
            base = j * 16
            pos = iot + base
            row = lax.shift_right_logical(pos * 5243, 20)
            qv = qbuf[pl.ds(base, 16)]
            t = qv * 100.0
            l = jnp.minimum(jnp.maximum(t.astype(jnp.int32) - 1, 0), 97)
            lf = (l + 1).astype(jnp.float32)
            ratio = (t - lf) * (1.0 / 1.1)
            al = row * _KP + l
            yl = plsc.load_gather(qobuf, [al])
            yr = plsc.load_gather(qobuf, [al + 1])
            center = yl + ratio * (yr - yl)
            fbv = fbbuf[pl.ds(base, 16)]
            tail = (qv < _KNOT0) | (qv > _KNOT98)
            obuf[pl.ds(base, 16)] = jnp.where(tail, fbv, center)

        pltpu.sync_copy(obuf, out_hbm.at[pl.ds(rbase * _Q, _CH * _Q)])
        return 0

    lax.fori_loop(0, _NCHUNK, chunk_body, 0)


def kernel(x, input_q, W, b):
    assert x.shape == (_B, _D) and input_q.shape == (_B, _Q)
    w_pad = jnp.zeros((_D, _KP), jnp.float32).at[:, :_K].set(W)
    b_pad = jnp.zeros((1, _KP), jnp.float32).at[0, :_K].set(b)

    qo, fb = pl.pallas_call(
        _tc_body,
        grid=(_B // _RB,),
        in_specs=[
            pl.BlockSpec((_RB, _D), lambda i: (i, 0)),
            pl.BlockSpec((_RB, _Q), lambda i: (i, 0)),
            pl.BlockSpec((_D, _KP), lambda i: (0, 0)),
            pl.BlockSpec((1, _KP), lambda i: (0, 0)),
        ],
        out_specs=[
            pl.BlockSpec((_RB, _KP), lambda i: (i, 0)),
            pl.BlockSpec((_RB, _Q), lambda i: (i, 0)),
        ],
        out_shape=[
            jax.ShapeDtypeStruct((_B, _KP), jnp.float32),
            jax.ShapeDtypeStruct((_B, _Q), jnp.float32),
        ],
    )(x, input_q, w_pad, b_pad)

    sc = functools.partial(
        pl.kernel,
        mesh=plsc.VectorSubcoreMesh(
            core_axis_name="c", subcore_axis_name="s",
            num_cores=2, num_subcores=16),
        out_type=jax.ShapeDtypeStruct((_B * _Q,), jnp.float32),
        scratch_types=[
            pltpu.VMEM((_CH * _Q,), jnp.float32),
            pltpu.VMEM((_CH * _Q,), jnp.float32),
            pltpu.VMEM((_CH * _KP,), jnp.float32),
            pltpu.VMEM((_CH * _Q,), jnp.float32),
        ],
        compiler_params=pltpu.CompilerParams(needs_layout_passes=False),
    )(_sc_kernel)

    out_flat = sc(input_q.reshape(-1), fb.reshape(-1), qo.reshape(-1))
    return out_flat.reshape(_B, _Q)


# speedup vs baseline: 3373.1922x; 1.7737x over previous
import functools

import numpy as np
import jax
import jax.numpy as jnp
from jax import lax
from jax.experimental import pallas as pl
from jax.experimental.pallas import tpu as pltpu
from jax.experimental.pallas import tpu_sc as plsc

_B = 16384
_D = 128
_Q = 200
_QP = 256
_K = 99
_KP = 128

_KNOT0 = float(np.float32(1.0) / np.float32(100.0))
_KNOT98 = float(np.float32(99.0) / np.float32(100.0))
_EPS = np.float32(1e-4)

_FK0 = np.float32(1.0) / np.float32(100.0)
_FK1 = np.float32(2.0) / np.float32(100.0)
_FK2 = np.float32(3.0) / np.float32(100.0)
_FK97 = np.float32(98.0) / np.float32(100.0)
_FK98 = np.float32(99.0) / np.float32(100.0)
_BRN = float(np.log(
    (np.float32(1.0) - _FK97 + _EPS) / (np.float32(1.0) - _FK98 + _EPS) + _EPS))
_BLN = float(np.log((_FK2 + _EPS) / (_FK1 + _EPS) + _EPS))
_LOG_1M_FK98 = float(np.log(np.float32(1.0) - _FK98))
_LOG_FK0 = float(np.log(_FK0))

_RB = 512
_CH = 64
_SIGN = np.int32(-2**31)


def _tc_body(x_ref, q_ref, w_ref, b_ref, qo_ref, a1_ref, a2_ref):
    x = x_ref[...]
    z = jnp.dot(x, w_ref[...], preferred_element_type=jnp.float32) + b_ref[...]
    s = jax.nn.softplus(z)
    rows_i = lax.broadcasted_iota(jnp.int32, (_KP, _KP), 0)
    cols_i = lax.broadcasted_iota(jnp.int32, (_KP, _KP), 1)
    tri = ((rows_i <= cols_i) & (rows_i < _K)).astype(jnp.float32)
    qo = jnp.dot(s, tri, preferred_element_type=jnp.float32)
    qo_ref[...] = qo

    q = q_ref[...]
    right_a = (qo[:, 97:98] - qo[:, 98:99]) / _BRN
    right_b = -right_a * _LOG_1M_FK98 + qo[:, 98:99]
    right_cal = jnp.log(1.0 - q) * right_a + right_b
    left_a = (qo[:, 1:2] - qo[:, 0:1]) / _BLN
    left_b = -left_a * _LOG_FK0 + qo[:, 0:1]
    left_cal = jnp.log(q) * left_a + left_b
    tail = (q < _KNOT0) | (q > _KNOT98)
    fb = jnp.where(q < _KNOT0, left_cal, right_cal)

    t = q * 100.0
    l = jnp.minimum(jnp.maximum(t.astype(jnp.int32) - 1, 0), 97)
    ratio = (t - (l + 1).astype(jnp.float32)) * (1.0 / 1.1)
    a1 = jnp.where(tail, fb, ratio)
    rloc = lax.broadcasted_iota(jnp.int32, (_RB, _Q), 0) & (_CH - 1)
    a2 = (rloc * _KP + l) | jnp.where(tail, _SIGN, 0)
    zf = jnp.zeros((_RB, _QP - _Q), jnp.float32)
    zi = jnp.zeros((_RB, _QP - _Q), jnp.int32)
    a1_ref[...] = jnp.concatenate([a1, zf], axis=1)
    a2_ref[...] = jnp.concatenate([a2, zi], axis=1)


_NW = 32
_RW = _B // _NW
_NCHUNK = _RW // _CH
_NV = 13


def _sc_kernel(a1_hbm, a2_hbm, qo_hbm, out_hbm,
               a1b0, a2b0, qob0, ob0, a1b1, a2b1, qob1, ob1,
               sin0, sin1, sout0, sout1):
    cid = lax.axis_index("c")
    sid = lax.axis_index("s")
    wid = sid * 2 + cid
    row0 = wid * _RW

    inbufs = ((a1b0, a2b0, qob0, sin0), (a1b1, a2b1, qob1, sin1))
    outbufs = ((ob0, sout0), (ob1, sout1))

    def start_in(ci, par):
        rbase = row0 + ci * _CH
        a1b, a2b, qob, sem = inbufs[par]
        pltpu.async_copy(a1_hbm.at[pl.ds(rbase, _CH)], a1b, sem)
        pltpu.async_copy(a2_hbm.at[pl.ds(rbase, _CH)], a2b, sem)
        pltpu.async_copy(qo_hbm.at[pl.ds(rbase, _CH)], qob, sem)

    def wait_in(ci, par):
        rbase = row0 + ci * _CH
        a1b, a2b, qob, sem = inbufs[par]
        pltpu.make_async_copy(a1_hbm.at[pl.ds(rbase, _CH)], a1b, sem).wait()
        pltpu.make_async_copy(a2_hbm.at[pl.ds(rbase, _CH)], a2b, sem).wait()
        pltpu.make_async_copy(qo_hbm.at[pl.ds(rbase, _CH)], qob, sem).wait()

    def start_out(ci, par):
        rbase = row0 + ci * _CH
        ob, sem = outbufs[par]
        pltpu.async_copy(ob, out_hbm.at[pl.ds(rbase, _CH)], sem)

    def wait_out(ci, par):
        rbase = row0 + ci * _CH
        ob, sem = outbufs[par]
        pltpu.make_async_copy(ob, out_hbm.at[pl.ds(rbase, _CH)], sem).wait()

    def compute(par):
        a1buf, a2buf, qobuf, _ = inbufs[par]
        obuf, _ = outbufs[par]

        @plsc.parallel_loop(0, _CH, unroll=2)
        def row_body(r):
            for v in range(_NV):
                c0 = v * 16
                a1 = a1buf[r, pl.ds(c0, 16)]
                a2 = a2buf[r, pl.ds(c0, 16)]
                tail = a2 < 0
                al = a2 & jnp.int32(0x7FFFFFFF)
                grow = lax.shift_right_logical(al, 7)
                gcol = al & 127
                yl = plsc.load_gather(qobuf, [grow, gcol])
                yr = plsc.load_gather(qobuf, [grow, gcol + 1])
                center = yl + a1 * (yr - yl)
                obuf[r, pl.ds(c0, 16)] = jnp.where(tail, a1, center)

    start_in(0, 0)

    def g_body(g, _):
        ci0 = 2 * g
        ci1 = 2 * g + 1
        start_in(ci1, 1)
        wait_in(ci0, 0)

        @pl.when(g > 0)
        def _w0():
            wait_out(ci0 - 2, 0)

        compute(0)
        start_out(ci0, 0)

        @pl.when(g < (_NCHUNK // 2) - 1)
        def _s0():
            start_in(ci1 + 1, 0)

        wait_in(ci1, 1)

        @pl.when(g > 0)
        def _w1():
            wait_out(ci1 - 2, 1)

        compute(1)
        start_out(ci1, 1)
        return 0

    lax.fori_loop(0, _NCHUNK // 2, g_body, 0)
    wait_out(_NCHUNK - 2, 0)
    wait_out(_NCHUNK - 1, 1)


def kernel(x, input_q, W, b):
    assert x.shape == (_B, _D) and input_q.shape == (_B, _Q)
    w_pad = jnp.zeros((_D, _KP), jnp.float32).at[:, :_K].set(W)
    b_pad = jnp.zeros((1, _KP), jnp.float32).at[0, :_K].set(b)

    qo, a1, a2 = pl.pallas_call(
        _tc_body,
        grid=(_B // _RB,),
        in_specs=[
            pl.BlockSpec((_RB, _D), lambda i: (i, 0)),
            pl.BlockSpec((_RB, _Q), lambda i: (i, 0)),
            pl.BlockSpec((_D, _KP), lambda i: (0, 0)),
            pl.BlockSpec((1, _KP), lambda i: (0, 0)),
        ],
        out_specs=[
            pl.BlockSpec((_RB, _KP), lambda i: (i, 0)),
            pl.BlockSpec((_RB, _QP), lambda i: (i, 0)),
            pl.BlockSpec((_RB, _QP), lambda i: (i, 0)),
        ],
        out_shape=[
            jax.ShapeDtypeStruct((_B, _KP), jnp.float32),
            jax.ShapeDtypeStruct((_B, _QP), jnp.float32),
            jax.ShapeDtypeStruct((_B, _QP), jnp.int32),
        ],
    )(x, input_q, w_pad, b_pad)

    sc = functools.partial(
        pl.kernel,
        mesh=plsc.VectorSubcoreMesh(
            core_axis_name="c", subcore_axis_name="s",
            num_cores=2, num_subcores=16),
        out_type=jax.ShapeDtypeStruct((_B, _QP), jnp.float32),
        scratch_types=[
            pltpu.VMEM((_CH, _QP), jnp.float32),
            pltpu.VMEM((_CH, _QP), jnp.int32),
            pltpu.VMEM((_CH, _KP), jnp.float32),
            pltpu.VMEM((_CH, _QP), jnp.float32),
            pltpu.VMEM((_CH, _QP), jnp.float32),
            pltpu.VMEM((_CH, _QP), jnp.int32),
            pltpu.VMEM((_CH, _KP), jnp.float32),
            pltpu.VMEM((_CH, _QP), jnp.float32),
            pltpu.SemaphoreType.DMA,
            pltpu.SemaphoreType.DMA,
            pltpu.SemaphoreType.DMA,
            pltpu.SemaphoreType.DMA,
        ],
        compiler_params=pltpu.CompilerParams(needs_layout_passes=False),
    )(_sc_kernel)

    out = sc(a1, a2, qo)
    return out[:, :_Q]
